# baseline (speedup 1.0000x reference)
import functools

import jax
import jax.numpy as jnp
from jax import lax
from jax.experimental import pallas as pl
from jax.experimental.pallas import tpu as pltpu
from jax.experimental.pallas import tpu_sc as plsc

VOCAB = 100000
EMB = 128
HID = 1024
SCALE = float(EMB) ** 0.5


_NW = 32
_CHUNK = 128
_NBUF = 6


def _sc_gather_body(table_hbm, idx_hbm, out_hbm, idx_all,
                    rows_0, rows_1, rows_2, rows_3, rows_4, rows_5,
                    gs_0, gs_1, gs_2, gs_3, gs_4, gs_5,
                    ws_0, ws_1, ws_2, ws_3, ws_4, ws_5, *, n_tokens):
    b_per_w = n_tokens // _NW
    n_chunks = b_per_w // _CHUNK
    wid = lax.axis_index("s") * 2 + lax.axis_index("c")
    base = wid * b_per_w

    row_bufs = [rows_0, rows_1, rows_2, rows_3, rows_4, rows_5]
    gsems = [gs_0, gs_1, gs_2, gs_3, gs_4, gs_5]
    wsems = [ws_0, ws_1, ws_2, ws_3, ws_4, ws_5]
    gcp = [None] * _NBUF
    wcp = [None] * _NBUF

    pltpu.sync_copy(idx_hbm.at[pl.ds(base, b_per_w)], idx_all)

    _LAG = 4
    for i in range(n_chunks + _LAG):
        if i < n_chunks:
            k = i % _NBUF
            if wcp[k] is not None:
                wcp[k].wait()
            gcp[k] = pltpu.async_copy(
                table_hbm.at[idx_all.at[pl.ds(i * _CHUNK, _CHUNK)]],
                row_bufs[k], gsems[k])
        j = i - _LAG
        if j >= 0:
            kp = j % _NBUF
            gcp[kp].wait()
            wcp[kp] = pltpu.async_copy(
                row_bufs[kp],
                out_hbm.at[pl.ds(base + j * _CHUNK, _CHUNK)],
                wsems[kp])
    for k in range(_NBUF):
        if wcp[k] is not None:
            wcp[k].wait()


def _sc_gather(table, idx_flat):
    n_tokens = idx_flat.shape[0]
    width = table.shape[1]
    mesh = plsc.VectorSubcoreMesh(core_axis_name="c", subcore_axis_name="s")
    return pl.kernel(
        functools.partial(_sc_gather_body, n_tokens=n_tokens),
        out_type=jax.ShapeDtypeStruct((n_tokens, width), table.dtype),
        mesh=mesh,
        scratch_types=(
            [pltpu.VMEM((n_tokens // _NW,), jnp.int32)]
            + [pltpu.VMEM((_CHUNK, width), table.dtype)] * _NBUF
            + [pltpu.SemaphoreType.DMA] * (2 * _NBUF)
        ),
    )(table, idx_flat)


_TOK_BLK = 4096


def _tc_project_body(emb_ref, ids_ref, w_ref, out_ref):
    rows = _TOK_BLK // 128
    mask = (ids_ref[...] != 0).astype(jnp.float32) * SCALE
    e = emb_ref[...].reshape(rows, 128, EMB) * mask[:, :, None]
    out_ref[...] = jnp.dot(e.reshape(_TOK_BLK, EMB), w_ref[...],
                           preferred_element_type=jnp.float32)


def _tc_project(gathered, ids_pack, w):
    n_tokens = gathered.shape[0]
    return pl.pallas_call(
        _tc_project_body,
        grid=(n_tokens // _TOK_BLK,),
        in_specs=[
            pl.BlockSpec((_TOK_BLK, EMB), lambda i: (i, 0)),
            pl.BlockSpec((_TOK_BLK // 128, 128), lambda i: (i, 0)),
            pl.BlockSpec((EMB, HID), lambda i: (0, 0)),
        ],
        out_specs=pl.BlockSpec((_TOK_BLK, HID), lambda i: (i, 0)),
        out_shape=jax.ShapeDtypeStruct((n_tokens, HID), jnp.float32),
    )(gathered, ids_pack, w)


def kernel(inputs, shared_weights, map_weights):
    b, l = inputs.shape
    idx_flat = inputs.reshape(-1)
    gathered = _sc_gather(shared_weights, idx_flat)
    out2d = _tc_project(gathered, idx_flat.reshape(-1, 128), map_weights)
    return out2d.reshape(b, l, HID)
